# Initial kernel scaffold; baseline (speedup 1.0000x reference)
import jax
import jax.numpy as jnp
from jax.experimental import pallas as pl


def kernel(x, W, b, attn_w, prelu_alpha, edges):
    raise NotImplementedError("write your pallas kernel here")


# speedup vs baseline: 4.4575x; 4.4575x over previous
import functools

import jax
import jax.numpy as jnp
from jax import lax
from jax.experimental import pallas as pl
from jax.experimental.pallas import tpu as pltpu
from jax.experimental.pallas import tpu_sc as plsc

N_IN = 10000
N_OUT = 10000
E = 320000
C = 128
DW = 144
NC = 2
NS = 16
NW = NC * NS
CHUNK = 128
ROWS = (E + CHUNK - 1) // CHUNK
ROWS_PAD = ((ROWS + NW - 1) // NW) * NW
RPT = ROWS_PAD // NW
E_PAD = ROWS_PAD * CHUNK
DUMP = N_OUT
ZPT = 626
ACC_ROWS = NS * ZPT
OPT = N_OUT // NS


def _prep_body(x_ref, w_ref, b_ref, attn_ref, alpha_ref, out_ref):
    xb = x_ref[...]
    wxT = lax.dot_general(xb, w_ref[...], (((0,), (1,)), ((), ())),
                          preferred_element_type=jnp.float32)
    wxT = wxT + b_ref[...]
    a = jnp.dot(wxT, attn_ref[...],
                preferred_element_type=jnp.float32)
    alpha = alpha_ref[...]
    g = jnp.exp(jnp.where(a >= 0.0, a, alpha * a))
    out_ref[:, 0:C] = wxT * g
    lane = lax.broadcasted_iota(jnp.int32, (N_IN, DW - C), 1)
    out_ref[:, C:DW] = jnp.where(lane == 0, g, 0.0)


def _prep(x, W, b, attn_w, prelu_alpha):
    return pl.pallas_call(
        _prep_body,
        out_shape=jax.ShapeDtypeStruct((N_IN, DW), jnp.float32),
    )(x, W, b.reshape(1, C), attn_w.reshape(C, 1),
      jnp.reshape(prelu_alpha, (1, 1)))


@functools.partial(
    pl.kernel,
    mesh=plsc.VectorSubcoreMesh(core_axis_name="c", subcore_axis_name="s"),
    out_type=jax.ShapeDtypeStruct((NC, N_OUT, DW), jnp.float32),
    compiler_params=pltpu.CompilerParams(use_tc_tiling_on_sc=False),
    scratch_types=[
        pltpu.VMEM((CHUNK,), jnp.int32),
        pltpu.VMEM((CHUNK,), jnp.int32),
        pltpu.VMEM((CHUNK, DW), jnp.float32),
        pltpu.VMEM_SHARED((ACC_ROWS, DW), jnp.float32),
        pltpu.SemaphoreType.DMA,
    ],
)
def _sc_scatter(src_hbm, tgt_hbm, h_hbm, z_hbm, out_hbm,
                src_v, tgt_v, rows_v, acc, sem):
    c = lax.axis_index("c")
    s = lax.axis_index("s")
    wid = c * NS + s
    pltpu.sync_copy(z_hbm, acc.at[pl.ds(s * ZPT, ZPT)])
    plsc.subcore_barrier()

    def body(r, carry):
        row = wid * RPT + r
        pltpu.sync_copy(src_hbm.at[row], src_v)
        pltpu.sync_copy(tgt_hbm.at[row], tgt_v)
        pltpu.async_copy(h_hbm.at[src_v], rows_v, sem).wait()
        pltpu.sync_copy(rows_v, acc.at[tgt_v], add=True)
        return carry

    lax.fori_loop(0, RPT, body, 0)
    plsc.subcore_barrier()
    pltpu.sync_copy(acc.at[pl.ds(s * OPT, OPT)],
                    out_hbm.at[c, pl.ds(s * OPT, OPT)])


def _fin_body(p0_ref, p1_ref, out_ref):
    t = p0_ref[...] + p1_ref[...]
    num = t[:, 0:C]
    den = t[:, C:C + 1]
    den = jnp.where(den == 0.0, 1.0, den)
    out_ref[...] = (num / den).T


def _fin(p0, p1):
    return pl.pallas_call(
        _fin_body,
        out_shape=jax.ShapeDtypeStruct((C, N_OUT), jnp.float32),
    )(p0, p1)


def kernel(x, W, b, attn_w, prelu_alpha, edges):
    e = edges.astype(jnp.int32)
    tgt = e[:, 0]
    src = e[:, 1]
    pad = E_PAD - E
    src2 = jnp.concatenate(
        [src, jnp.zeros((pad,), jnp.int32)]).reshape(ROWS_PAD, CHUNK)
    tgt2 = jnp.concatenate(
        [tgt, jnp.full((pad,), DUMP, jnp.int32)]).reshape(ROWS_PAD, CHUNK)
    haug = _prep(x, W, b, attn_w, prelu_alpha)
    zeros = jnp.zeros((ZPT, DW), jnp.float32)
    partials = _sc_scatter(src2, tgt2, haug, zeros)
    return _fin(partials[0], partials[1])


# speedup vs baseline: 5.6452x; 1.2665x over previous
import functools

import jax
import jax.numpy as jnp
from jax import lax
from jax.experimental import pallas as pl
from jax.experimental.pallas import tpu as pltpu
from jax.experimental.pallas import tpu_sc as plsc

N_IN = 10000
N_OUT = 10000
E = 320000
C = 128
DW = 144
NC = 2
NS = 16
NW = NC * NS
CHUNK = 64
RPT = 158
NPAIR = RPT // 2
ROWS_PAD = NW * RPT
E_PAD = ROWS_PAD * CHUNK
N_TAB = N_IN + 16
ZSRC = N_IN
ACC_ROWS = N_OUT
ZPT = N_OUT // NS


def _prep_body(x_ref, w_ref, b_ref, attn_ref, alpha_ref, out_ref):
    xb = x_ref[...]
    wxT = lax.dot_general(xb, w_ref[...], (((0,), (1,)), ((), ())),
                          preferred_element_type=jnp.float32)
    wxT = wxT + b_ref[...]
    a = jnp.dot(wxT, attn_ref[...],
                preferred_element_type=jnp.float32)
    alpha = alpha_ref[...]
    g = jnp.exp(jnp.where(a >= 0.0, a, alpha * a))
    out_ref[0:N_IN, 0:C] = wxT * g
    lane = lax.broadcasted_iota(jnp.int32, (N_IN, DW - C), 1)
    out_ref[0:N_IN, C:DW] = jnp.where(lane == 0, g, 0.0)
    out_ref[N_IN:N_TAB, :] = jnp.zeros((N_TAB - N_IN, DW), jnp.float32)


def _prep(x, W, b, attn_w, prelu_alpha):
    return pl.pallas_call(
        _prep_body,
        out_shape=jax.ShapeDtypeStruct((N_TAB, DW), jnp.float32),
    )(x, W, b.reshape(1, C), attn_w.reshape(C, 1),
      jnp.reshape(prelu_alpha, (1, 1)))


@functools.partial(
    pl.kernel,
    mesh=plsc.VectorSubcoreMesh(core_axis_name="c", subcore_axis_name="s"),
    out_type=jax.ShapeDtypeStruct((NC, N_OUT, DW), jnp.float32),
    compiler_params=pltpu.CompilerParams(use_tc_tiling_on_sc=False),
    scratch_types=[
        pltpu.VMEM((RPT, CHUNK), jnp.int32),
        pltpu.VMEM((RPT, CHUNK), jnp.int32),
        pltpu.VMEM((2, CHUNK, DW), jnp.float32),
        pltpu.VMEM_SHARED((ACC_ROWS, DW), jnp.float32),
        pltpu.SemaphoreType.DMA,
        pltpu.SemaphoreType.DMA,
    ],
)
def _sc_scatter(src_hbm, tgt_hbm, h_hbm, z_hbm, out_hbm,
                src_v, tgt_v, rows_v, acc, sem_a, sem_b):
    c = lax.axis_index("c")
    s = lax.axis_index("s")
    wid = c * NS + s
    base = wid * RPT
    pltpu.sync_copy(src_hbm.at[pl.ds(base, RPT)], src_v)
    pltpu.sync_copy(tgt_hbm.at[pl.ds(base, RPT)], tgt_v)
    pltpu.sync_copy(z_hbm, acc.at[pl.ds(s * ZPT, ZPT)])
    plsc.subcore_barrier()

    pltpu.async_copy(h_hbm.at[src_v.at[0]], rows_v.at[0], sem_a)

    def body(p, carry):
        r0 = 2 * p
        r1 = r0 + 1
        pltpu.async_copy(h_hbm.at[src_v.at[r1]], rows_v.at[1], sem_b)
        pltpu.make_async_copy(h_hbm.at[src_v.at[r0]], rows_v.at[0],
                              sem_a).wait()
        pltpu.sync_copy(rows_v.at[0], acc.at[tgt_v.at[r0]], add=True)

        @pl.when(p < NPAIR - 1)
        def _():
            pltpu.async_copy(h_hbm.at[src_v.at[r0 + 2]], rows_v.at[0], sem_a)

        pltpu.make_async_copy(h_hbm.at[src_v.at[r1]], rows_v.at[1],
                              sem_b).wait()
        pltpu.sync_copy(rows_v.at[1], acc.at[tgt_v.at[r1]], add=True)
        return carry

    lax.fori_loop(0, NPAIR, body, 0)
    plsc.subcore_barrier()
    pltpu.sync_copy(acc.at[pl.ds(s * ZPT, ZPT)],
                    out_hbm.at[c, pl.ds(s * ZPT, ZPT)])


def _fin_body(p0_ref, p1_ref, out_ref):
    t = p0_ref[...] + p1_ref[...]
    num = t[:, 0:C]
    den = t[:, C:C + 1]
    den = jnp.where(den == 0.0, 1.0, den)
    out_ref[...] = (num / den).T


def _fin(p0, p1):
    return pl.pallas_call(
        _fin_body,
        out_shape=jax.ShapeDtypeStruct((C, N_OUT), jnp.float32),
    )(p0, p1)


def kernel(x, W, b, attn_w, prelu_alpha, edges):
    e = edges.astype(jnp.int32)
    tgt = e[:, 0]
    src = e[:, 1]
    pad = E_PAD - E
    src2 = jnp.concatenate(
        [src, jnp.full((pad,), ZSRC, jnp.int32)]).reshape(ROWS_PAD, CHUNK)
    tgt2 = jnp.concatenate(
        [tgt, jnp.zeros((pad,), jnp.int32)]).reshape(ROWS_PAD, CHUNK)
    haug = _prep(x, W, b, attn_w, prelu_alpha)
    zeros = jnp.zeros((ZPT, DW), jnp.float32)
    partials = _sc_scatter(src2, tgt2, haug, zeros)
    return _fin(partials[0], partials[1])


# speedup vs baseline: 6.5173x; 1.1545x over previous
import functools

import jax
import jax.numpy as jnp
from jax import lax
from jax.experimental import pallas as pl
from jax.experimental.pallas import tpu as pltpu
from jax.experimental.pallas import tpu_sc as plsc

N_IN = 10000
N_OUT = 10000
E = 320000
C = 128
DW = 144
NC = 2
NS = 16
NW = NC * NS
CHUNK = 48
RPT = 210
NTRI = RPT // 3
ROWS_PAD = NW * RPT
E_PAD = ROWS_PAD * CHUNK
N_TAB = N_IN + 16
ZSRC = N_IN
ACC_ROWS = N_OUT
ZPT = N_OUT // NS


def _prep_body(x_ref, w_ref, b_ref, attn_ref, alpha_ref, out_ref):
    xb = x_ref[...]
    wxT = lax.dot_general(xb, w_ref[...], (((0,), (1,)), ((), ())),
                          preferred_element_type=jnp.float32)
    wxT = wxT + b_ref[...]
    a = jnp.dot(wxT, attn_ref[...],
                preferred_element_type=jnp.float32)
    alpha = alpha_ref[...]
    g = jnp.exp(jnp.where(a >= 0.0, a, alpha * a))
    out_ref[0:N_IN, 0:C] = wxT * g
    lane = lax.broadcasted_iota(jnp.int32, (N_IN, DW - C), 1)
    out_ref[0:N_IN, C:DW] = jnp.where(lane == 0, g, 0.0)
    out_ref[N_IN:N_TAB, :] = jnp.zeros((N_TAB - N_IN, DW), jnp.float32)


def _prep(x, W, b, attn_w, prelu_alpha):
    return pl.pallas_call(
        _prep_body,
        out_shape=jax.ShapeDtypeStruct((N_TAB, DW), jnp.float32),
    )(x, W, b.reshape(1, C), attn_w.reshape(C, 1),
      jnp.reshape(prelu_alpha, (1, 1)))


@functools.partial(
    pl.kernel,
    mesh=plsc.VectorSubcoreMesh(core_axis_name="c", subcore_axis_name="s"),
    out_type=jax.ShapeDtypeStruct((NC, N_OUT, DW), jnp.float32),
    compiler_params=pltpu.CompilerParams(use_tc_tiling_on_sc=False),
    scratch_types=[
        pltpu.VMEM((RPT, CHUNK), jnp.int32),
        pltpu.VMEM((RPT, CHUNK), jnp.int32),
        pltpu.VMEM((3, CHUNK, DW), jnp.float32),
        pltpu.VMEM_SHARED((ACC_ROWS, DW), jnp.float32),
        [pltpu.SemaphoreType.DMA] * 3,
        [pltpu.SemaphoreType.DMA] * 3,
    ],
)
def _sc_scatter(src_hbm, tgt_hbm, h_hbm, z_hbm, out_hbm,
                src_v, tgt_v, rows_v, acc, sem_g, sem_s):
    c = lax.axis_index("c")
    s = lax.axis_index("s")
    wid = c * NS + s
    base = wid * RPT
    pltpu.sync_copy(src_hbm.at[pl.ds(base, RPT)], src_v)
    pltpu.sync_copy(tgt_hbm.at[pl.ds(base, RPT)], tgt_v)
    pltpu.sync_copy(z_hbm, acc.at[pl.ds(s * ZPT, ZPT)])
    plsc.subcore_barrier()

    pltpu.async_copy(h_hbm.at[src_v.at[0]], rows_v.at[0], sem_g[0])
    pltpu.async_copy(h_hbm.at[src_v.at[1]], rows_v.at[1], sem_g[1])

    def step(r, b):
        pltpu.make_async_copy(h_hbm.at[src_v.at[r]], rows_v.at[b],
                              sem_g[b]).wait()
        pltpu.async_copy(rows_v.at[b], acc.at[tgt_v.at[r]], sem_s[b],
                         add=True)
        b2 = (b + 2) % 3

        @pl.when(r >= 1)
        def _():
            pltpu.make_async_copy(rows_v.at[b2], acc.at[tgt_v.at[r - 1]],
                                  sem_s[b2]).wait()

        @pl.when(r + 2 < RPT)
        def _():
            pltpu.async_copy(h_hbm.at[src_v.at[r + 2]], rows_v.at[b2],
                             sem_g[b2])

    def body(i, carry):
        r0 = 3 * i
        step(r0, 0)
        step(r0 + 1, 1)
        step(r0 + 2, 2)
        return carry

    lax.fori_loop(0, NTRI, body, 0)
    pltpu.make_async_copy(rows_v.at[(RPT - 1) % 3],
                          acc.at[tgt_v.at[RPT - 1]],
                          sem_s[(RPT - 1) % 3]).wait()
    plsc.subcore_barrier()
    pltpu.sync_copy(acc.at[pl.ds(s * ZPT, ZPT)],
                    out_hbm.at[c, pl.ds(s * ZPT, ZPT)])


def _fin_body(p0_ref, p1_ref, out_ref):
    t = p0_ref[...] + p1_ref[...]
    num = t[:, 0:C]
    den = t[:, C:C + 1]
    den = jnp.where(den == 0.0, 1.0, den)
    out_ref[...] = (num / den).T


def _fin(p0, p1):
    return pl.pallas_call(
        _fin_body,
        out_shape=jax.ShapeDtypeStruct((C, N_OUT), jnp.float32),
    )(p0, p1)


def kernel(x, W, b, attn_w, prelu_alpha, edges):
    e = edges.astype(jnp.int32)
    tgt = e[:, 0]
    src = e[:, 1]
    pad = E_PAD - E
    src2 = jnp.concatenate(
        [src, jnp.full((pad,), ZSRC, jnp.int32)]).reshape(ROWS_PAD, CHUNK)
    tgt2 = jnp.concatenate(
        [tgt, jnp.zeros((pad,), jnp.int32)]).reshape(ROWS_PAD, CHUNK)
    haug = _prep(x, W, b, attn_w, prelu_alpha)
    zeros = jnp.zeros((ZPT, DW), jnp.float32)
    partials = _sc_scatter(src2, tgt2, haug, zeros)
    return _fin(partials[0], partials[1])


# speedup vs baseline: 7.2714x; 1.1157x over previous
import functools

import jax
import jax.numpy as jnp
from jax import lax
from jax.experimental import pallas as pl
from jax.experimental.pallas import tpu as pltpu
from jax.experimental.pallas import tpu_sc as plsc

N_IN = 10000
N_OUT = 10000
E = 320000
C = 128
DW = 144
NC = 2
NS = 16
NW = NC * NS
CHUNK = 48
RPT0 = 267
RPT1 = 153
RPTMAX = max(RPT0, RPT1)
ROWS_PAD = NS * (RPT0 + RPT1)
E_PAD = ROWS_PAD * CHUNK
N_TAB = N_IN + 16
ZSRC = N_IN
ACC_ROWS = N_OUT
ZPT = N_OUT // NS


def _prep_body(x_ref, w_ref, b_ref, attn_ref, alpha_ref, out_ref):
    xb = x_ref[...]
    wxT = lax.dot_general(xb, w_ref[...], (((0,), (1,)), ((), ())),
                          preferred_element_type=jnp.float32)
    wxT = wxT + b_ref[...]
    a = jnp.dot(wxT, attn_ref[...],
                preferred_element_type=jnp.float32)
    alpha = alpha_ref[...]
    g = jnp.exp(jnp.where(a >= 0.0, a, alpha * a))
    out_ref[0:N_IN, 0:C] = wxT * g
    lane = lax.broadcasted_iota(jnp.int32, (N_IN, DW - C), 1)
    out_ref[0:N_IN, C:DW] = jnp.where(lane == 0, g, 0.0)
    out_ref[N_IN:N_TAB, :] = jnp.zeros((N_TAB - N_IN, DW), jnp.float32)


def _prep(x, W, b, attn_w, prelu_alpha):
    return pl.pallas_call(
        _prep_body,
        out_shape=jax.ShapeDtypeStruct((N_TAB, DW), jnp.float32),
    )(x, W, b.reshape(1, C), attn_w.reshape(C, 1),
      jnp.reshape(prelu_alpha, (1, 1)))


@functools.partial(
    pl.kernel,
    mesh=plsc.VectorSubcoreMesh(core_axis_name="c", subcore_axis_name="s"),
    out_type=jax.ShapeDtypeStruct((NC, N_OUT, DW), jnp.float32),
    compiler_params=pltpu.CompilerParams(use_tc_tiling_on_sc=False),
    scratch_types=[
        pltpu.VMEM((RPTMAX, CHUNK), jnp.int32),
        pltpu.VMEM((3, CHUNK), jnp.int32),
        pltpu.VMEM((3, CHUNK), jnp.int32),
        pltpu.VMEM((3, CHUNK, DW), jnp.float32),
        pltpu.VMEM_SHARED((ACC_ROWS, DW), jnp.float32),
        [pltpu.SemaphoreType.DMA] * 3,
        [pltpu.SemaphoreType.DMA] * 3,
    ],
)
def _sc_scatter(pk_hbm, h_hbm, z_hbm, out_hbm,
                pk_v, src_v, tgt_v, rows_v, acc, sem_g, sem_s):
    c = lax.axis_index("c")
    s = lax.axis_index("s")
    rpt = jnp.where(c == 0, RPT0, RPT1)

    @pl.when(c == 0)
    def _():
        pltpu.sync_copy(pk_hbm.at[pl.ds(s * RPT0, RPT0)],
                        pk_v.at[pl.ds(0, RPT0)])

    @pl.when(c == 1)
    def _():
        pltpu.sync_copy(pk_hbm.at[pl.ds(NS * RPT0 + s * RPT1, RPT1)],
                        pk_v.at[pl.ds(0, RPT1)])

    pltpu.sync_copy(z_hbm, acc.at[pl.ds(s * ZPT, ZPT)])
    plsc.subcore_barrier()

    def unpack(r, b):
        for q in range(CHUNK // 16):
            pk = pk_v[r, pl.ds(q * 16, 16)]
            src_v[b, pl.ds(q * 16, 16)] = pk & 0xFFFF
            tgt_v[b, pl.ds(q * 16, 16)] = lax.shift_right_logical(pk, 16)

    unpack(0, 0)
    pltpu.async_copy(h_hbm.at[src_v.at[0]], rows_v.at[0], sem_g[0])
    unpack(1, 1)
    pltpu.async_copy(h_hbm.at[src_v.at[1]], rows_v.at[1], sem_g[1])

    def step(r, b):
        pltpu.make_async_copy(h_hbm.at[src_v.at[b]], rows_v.at[b],
                              sem_g[b]).wait()
        pltpu.async_copy(rows_v.at[b], acc.at[tgt_v.at[b]], sem_s[b],
                         add=True)
        b2 = (b + 2) % 3

        @pl.when(r >= 1)
        def _():
            pltpu.make_async_copy(rows_v.at[b2], acc.at[tgt_v.at[b2]],
                                  sem_s[b2]).wait()

        @pl.when(r + 2 < rpt)
        def _():
            unpack(r + 2, b2)
            pltpu.async_copy(h_hbm.at[src_v.at[b2]], rows_v.at[b2],
                             sem_g[b2])

    def body(i, carry):
        r0 = 3 * i
        step(r0, 0)
        step(r0 + 1, 1)
        step(r0 + 2, 2)
        return carry

    lax.fori_loop(0, rpt // 3, body, 0)
    pltpu.make_async_copy(rows_v.at[2], acc.at[tgt_v.at[2]],
                          sem_s[2]).wait()
    plsc.subcore_barrier()
    pltpu.sync_copy(acc.at[pl.ds(s * ZPT, ZPT)],
                    out_hbm.at[c, pl.ds(s * ZPT, ZPT)])


def _fin_body(p0_ref, p1_ref, out_ref):
    t = p0_ref[...] + p1_ref[...]
    num = t[:, 0:C]
    den = t[:, C:C + 1]
    den = jnp.where(den == 0.0, 1.0, den)
    out_ref[...] = (num / den).T


def _fin(p0, p1):
    return pl.pallas_call(
        _fin_body,
        out_shape=jax.ShapeDtypeStruct((C, N_OUT), jnp.float32),
    )(p0, p1)


def kernel(x, W, b, attn_w, prelu_alpha, edges):
    e = edges.astype(jnp.int32)
    tgt = e[:, 0]
    src = e[:, 1]
    pad = E_PAD - E
    packed = tgt * 65536 + src
    pk = jnp.concatenate(
        [packed, jnp.full((pad,), ZSRC, jnp.int32)]).reshape(ROWS_PAD, CHUNK)
    haug = _prep(x, W, b, attn_w, prelu_alpha)
    zeros = jnp.zeros((ZPT, DW), jnp.float32)
    partials = _sc_scatter(pk, haug, zeros)
    return _fin(partials[0], partials[1])
